# baseline (speedup 1.0000x reference)
import functools

import jax
import jax.numpy as jnp
from jax import lax
from jax.experimental import pallas as pl
from jax.experimental.pallas import tpu as pltpu
from jax.experimental.pallas import tpu_sc as plsc

VOCAB = 1000000
BATCH = 16384
UNITS = 10

_NC = 2
_NS = 16
_NW = _NC * _NS
_BPW = BATCH // _NW
_ICH = 128
_KCH = _BPW // _ICH
_LANES = 16

_mesh = plsc.VectorSubcoreMesh(
    core_axis_name="c", subcore_axis_name="s", num_cores=_NC, num_subcores=_NS
)


@functools.partial(
    pl.kernel,
    out_type=jax.ShapeDtypeStruct((BATCH * UNITS,), jnp.float32),
    mesh=_mesh,
    scratch_types=[
        pltpu.VMEM((_KCH, _ICH), jnp.int32),
        pltpu.VMEM((_BPW,), jnp.int32),
        pltpu.VMEM((2, _LANES), jnp.float32),
        pltpu.VMEM((UNITS * _BPW,), jnp.float32),
        pltpu.SemaphoreType.DMA,
    ],
)
def _lookup_affine(table_h, idx_h, wb_h, out_h, idx_v, ids_v, wb_v, out_v, sem):
    wid = lax.axis_index("s") * _NC + lax.axis_index("c")
    pltpu.sync_copy(idx_h.at[wid], idx_v)
    pltpu.sync_copy(wb_h, wb_v)
    copies = [
        pltpu.async_copy(
            table_h.at[idx_v.at[k]], ids_v.at[pl.ds(k * _ICH, _ICH)], sem
        )
        for k in range(_KCH)
    ]
    for c in copies:
        c.wait()
    wrow = wb_v[0]
    brow = wb_v[1]
    ws = [wrow[j] for j in range(UNITS)]
    bs = [brow[j] for j in range(UNITS)]
    for i in range(_BPW // _LANES):
        v = ids_v[pl.ds(i * _LANES, _LANES)].astype(jnp.float32)
        for j in range(UNITS):
            out_v[pl.ds(j * _BPW + i * _LANES, _LANES)] = v * ws[j] + bs[j]
    pltpu.sync_copy(out_v, out_h.at[pl.ds(wid * UNITS * _BPW, UNITS * _BPW)])


def kernel(inputs, lookup_table, W, b):
    idx = inputs.reshape(-1).astype(jnp.int32).reshape(_NW, _KCH, _ICH)
    wb = jnp.zeros((2, _LANES), jnp.float32)
    wb = wb.at[0, :UNITS].set(W[0].astype(jnp.float32))
    wb = wb.at[1, :UNITS].set(b.astype(jnp.float32))
    out = _lookup_affine(lookup_table, idx, wb)
    return out.reshape(BATCH, UNITS)


# speedup vs baseline: 1.5921x; 1.5921x over previous
import functools

import jax
import jax.numpy as jnp
from jax import lax
from jax.experimental import pallas as pl
from jax.experimental.pallas import tpu as pltpu
from jax.experimental.pallas import tpu_sc as plsc

VOCAB = 1000000
BATCH = 16384
UNITS = 10

_NC = 2
_NS = 16
_NW = _NC * _NS
_BPW = BATCH // _NW
_ICH = 128
_KCH = _BPW // _ICH
_LANES = 16

_mesh = plsc.VectorSubcoreMesh(
    core_axis_name="c", subcore_axis_name="s", num_cores=_NC, num_subcores=_NS
)


@functools.partial(
    pl.kernel,
    out_type=jax.ShapeDtypeStruct((_NW, UNITS, _BPW), jnp.float32),
    mesh=_mesh,
    scratch_types=[
        pltpu.VMEM((_KCH, _ICH), jnp.int32),
        pltpu.VMEM((_BPW,), jnp.int32),
        pltpu.VMEM((2, _LANES), jnp.float32),
        pltpu.VMEM((UNITS, _BPW), jnp.float32),
        pltpu.SemaphoreType.DMA,
        pltpu.SemaphoreType.DMA,
        pltpu.SemaphoreType.DMA,
        pltpu.SemaphoreType.DMA,
        pltpu.SemaphoreType.DMA,
    ],
)
def _lookup_affine(
    table_h, idx_h, w_h, b_h, out_h, idx_v, ids_v, wb_v, out_v, wb_sem, *sems
):
    wid = lax.axis_index("s") * _NC + lax.axis_index("c")
    wcp = pltpu.async_copy(w_h, wb_v.at[0, pl.ds(0, UNITS)], wb_sem)
    bcp = pltpu.async_copy(b_h, wb_v.at[1, pl.ds(0, UNITS)], wb_sem)
    pltpu.sync_copy(idx_h.at[wid], idx_v)
    copies = [
        pltpu.async_copy(
            table_h.at[idx_v.at[k]], ids_v.at[pl.ds(k * _ICH, _ICH)], sems[k]
        )
        for k in range(_KCH)
    ]
    wcp.wait()
    bcp.wait()
    wrow = wb_v[0]
    brow = wb_v[1]
    ws = [wrow[j] for j in range(UNITS)]
    bs = [brow[j] for j in range(UNITS)]
    for k in range(_KCH):
        copies[k].wait()
        for i in range(k * _ICH // _LANES, (k + 1) * _ICH // _LANES):
            v = ids_v[pl.ds(i * _LANES, _LANES)].astype(jnp.float32)
            for j in range(UNITS):
                out_v[j, pl.ds(i * _LANES, _LANES)] = v * ws[j] + bs[j]
    pltpu.sync_copy(out_v, out_h.at[wid])


def kernel(inputs, lookup_table, W, b):
    idx = inputs.reshape(-1).astype(jnp.int32).reshape(_NW, _KCH, _ICH)
    out = _lookup_affine(
        lookup_table, idx, W.reshape(UNITS).astype(jnp.float32),
        b.astype(jnp.float32)
    )
    return out.transpose(0, 2, 1).reshape(BATCH, UNITS)
